# baseline (speedup 1.0000x reference)
import jax
import jax.numpy as jnp
from jax.experimental import pallas as pl


def kernel(item_seqs, emb):
    batch, seq_len = item_seqs.shape
    hidden = emb.shape[1]
    bb = 256

    def body(emb_ref, out_ref):
        out_ref[...] = jnp.broadcast_to(
            emb_ref[...][None], (bb, seq_len, hidden)
        )

    out = pl.pallas_call(
        body,
        grid=(batch // bb,),
        in_specs=[pl.BlockSpec((seq_len, hidden), lambda i: (0, 0))],
        out_specs=pl.BlockSpec((bb, seq_len, hidden), lambda i: (i, 0, 0)),
        out_shape=jax.ShapeDtypeStruct((batch, seq_len, hidden), jnp.float32),
    )(emb[:seq_len])
    return out


# speedup vs baseline: 1.0162x; 1.0162x over previous
import jax
import jax.numpy as jnp
from jax.experimental import pallas as pl
from jax.experimental.pallas import tpu as pltpu


def kernel(item_seqs, emb):
    batch, seq_len = item_seqs.shape
    hidden = emb.shape[1]
    rep = 64
    n_chunks = batch // rep
    rows = rep * seq_len

    def body(emb_ref, out_ref, buf, sem):
        buf[...] = jnp.broadcast_to(
            emb_ref[...][None], (rep, seq_len, hidden)
        ).reshape(rows, hidden)
        handles = [
            pltpu.make_async_copy(
                buf, out_ref.at[pl.ds(c * rows, rows)], sem
            )
            for c in range(n_chunks)
        ]
        for h in handles:
            h.start()
        for h in handles:
            h.wait()

    out = pl.pallas_call(
        body,
        in_specs=[pl.BlockSpec(memory_space=pltpu.VMEM)],
        out_specs=pl.BlockSpec(memory_space=pl.ANY),
        out_shape=jax.ShapeDtypeStruct((batch * seq_len, hidden), jnp.float32),
        scratch_shapes=[
            pltpu.VMEM((rows, hidden), jnp.float32),
            pltpu.SemaphoreType.DMA,
        ],
    )(emb[:seq_len])
    return out.reshape(batch, seq_len, hidden)
